# Initial kernel scaffold; baseline (speedup 1.0000x reference)
import jax
import jax.numpy as jnp
from jax.experimental import pallas as pl


def kernel(x, table):
    raise NotImplementedError("write your pallas kernel here")


# speedup vs baseline: 2.6254x; 2.6254x over previous
import functools

import jax
import jax.numpy as jnp
from jax import lax
from jax.experimental import pallas as pl
from jax.experimental.pallas import tpu as pltpu
from jax.experimental.pallas import tpu_sc as plsc

B = 16384
L = 50
D = 32
NC = 2
NS = 16
NW = NC * NS
SAMPLES_PER_CHUNK = 2
IDX_PER_CHUNK = SAMPLES_PER_CHUNK * L
SW = B // NW
CW = SW // SAMPLES_PER_CHUNK
INV_L = 1.0 / L


def _body(x_hbm, table_hbm, out_hbm, idx_v, rows0, rows1, out_v, sem0, sem1):
    wid = lax.axis_index("s") * NC + lax.axis_index("c")

    pltpu.sync_copy(x_hbm.at[pl.ds(wid * CW, CW)], idx_v)

    def start(c, rows, sem):
        pltpu.async_copy(table_hbm.at[idx_v.at[c]], rows, sem)

    def wait(c, rows, sem):
        pltpu.make_async_copy(table_hbm.at[idx_v.at[c]], rows, sem).wait()

    def reduce_chunk(rows, c):
        for s in range(SAMPLES_PER_CHUNK):
            acc0 = jnp.zeros((16,), jnp.float32)
            acc1 = jnp.zeros((16,), jnp.float32)
            for r in range(L):
                acc0 = acc0 + rows[s * L + r, pl.ds(0, 16)]
                acc1 = acc1 + rows[s * L + r, pl.ds(16, 16)]
            out_v[SAMPLES_PER_CHUNK * c + s, pl.ds(0, 16)] = acc0 * INV_L
            out_v[SAMPLES_PER_CHUNK * c + s, pl.ds(16, 16)] = acc1 * INV_L

    start(0, rows0, sem0)
    start(1, rows1, sem1)

    @pl.loop(0, CW // 2)
    def _(i):
        c0 = 2 * i
        c1 = 2 * i + 1
        wait(c0, rows0, sem0)
        reduce_chunk(rows0, c0)

        @pl.when(c0 + 2 < CW)
        def _():
            start(c0 + 2, rows0, sem0)

        wait(c1, rows1, sem1)
        reduce_chunk(rows1, c1)

        @pl.when(c1 + 2 < CW)
        def _():
            start(c1 + 2, rows1, sem1)

    pltpu.sync_copy(out_v, out_hbm.at[pl.ds(wid * SW, SW)])


@jax.jit
def kernel(x, table):
    mesh = plsc.VectorSubcoreMesh(
        core_axis_name="c", subcore_axis_name="s",
        num_cores=NC, num_subcores=NS,
    )
    x2 = x.reshape(B * L // IDX_PER_CHUNK, IDX_PER_CHUNK).astype(jnp.int32)
    run = pl.kernel(
        _body,
        out_type=jax.ShapeDtypeStruct((B, D), jnp.float32),
        mesh=mesh,
        scratch_types=[
            pltpu.VMEM((CW, IDX_PER_CHUNK), jnp.int32),
            pltpu.VMEM((IDX_PER_CHUNK, D), jnp.float32),
            pltpu.VMEM((IDX_PER_CHUNK, D), jnp.float32),
            pltpu.VMEM((SW, D), jnp.float32),
            pltpu.SemaphoreType.DMA,
            pltpu.SemaphoreType.DMA,
        ],
        compiler_params=pltpu.CompilerParams(use_tc_tiling_on_sc=False),
    )
    return run(x2, table)


# speedup vs baseline: 2.7490x; 1.0470x over previous
import functools

import jax
import jax.numpy as jnp
from jax import lax
from jax.experimental import pallas as pl
from jax.experimental.pallas import tpu as pltpu
from jax.experimental.pallas import tpu_sc as plsc

B = 16384
L = 50
D = 32
NC = 2
NS = 16
NW = NC * NS
SAMPLES_PER_CHUNK = 2
IDX_PER_CHUNK = SAMPLES_PER_CHUNK * L
SW = B // NW
CW = SW // SAMPLES_PER_CHUNK
INV_L = 1.0 / L


NBUF = 8


def _body(x_hbm, table_hbm, out_hbm, idx_v, rows_b, out_v, sems):
    wid = lax.axis_index("s") * NC + lax.axis_index("c")

    pltpu.sync_copy(x_hbm.at[pl.ds(wid * CW, CW)], idx_v)

    def start(c, b):
        pltpu.async_copy(table_hbm.at[idx_v.at[c]], rows_b.at[b], sems.at[b])

    def wait(c, b):
        pltpu.make_async_copy(
            table_hbm.at[idx_v.at[c]], rows_b.at[b], sems.at[b]).wait()

    def reduce_chunk(b, c):
        rows = rows_b.at[b]
        for s in range(SAMPLES_PER_CHUNK):
            acc0a = jnp.zeros((16,), jnp.float32)
            acc0b = jnp.zeros((16,), jnp.float32)
            acc1a = jnp.zeros((16,), jnp.float32)
            acc1b = jnp.zeros((16,), jnp.float32)
            for r in range(0, L, 2):
                acc0a = acc0a + rows[s * L + r, pl.ds(0, 16)]
                acc1a = acc1a + rows[s * L + r, pl.ds(16, 16)]
                acc0b = acc0b + rows[s * L + r + 1, pl.ds(0, 16)]
                acc1b = acc1b + rows[s * L + r + 1, pl.ds(16, 16)]
            out_v[SAMPLES_PER_CHUNK * c + s, pl.ds(0, 16)] = (
                acc0a + acc0b) * INV_L
            out_v[SAMPLES_PER_CHUNK * c + s, pl.ds(16, 16)] = (
                acc1a + acc1b) * INV_L

    for b in range(NBUF):
        start(b, b)

    @pl.loop(0, CW // NBUF)
    def _(i):
        base = i * NBUF
        for b in range(NBUF):
            c = base + b
            wait(c, b)
            reduce_chunk(b, c)

            @pl.when(c + NBUF < CW)
            def _():
                start(c + NBUF, b)

    pltpu.sync_copy(out_v, out_hbm.at[pl.ds(wid * SW, SW)])


@jax.jit
def kernel(x, table):
    mesh = plsc.VectorSubcoreMesh(
        core_axis_name="c", subcore_axis_name="s",
        num_cores=NC, num_subcores=NS,
    )
    x2 = x.reshape(B * L // IDX_PER_CHUNK, IDX_PER_CHUNK).astype(jnp.int32)
    run = pl.kernel(
        _body,
        out_type=jax.ShapeDtypeStruct((B, D), jnp.float32),
        mesh=mesh,
        scratch_types=[
            pltpu.VMEM((CW, IDX_PER_CHUNK), jnp.int32),
            pltpu.VMEM((NBUF, IDX_PER_CHUNK, D), jnp.float32),
            pltpu.VMEM((SW, D), jnp.float32),
            pltpu.SemaphoreType.DMA((NBUF,)),
        ],
        compiler_params=pltpu.CompilerParams(use_tc_tiling_on_sc=False),
    )
    return run(x2, table)
